# Initial kernel scaffold; baseline (speedup 1.0000x reference)
import jax
import jax.numpy as jnp
from jax.experimental import pallas as pl


def kernel(x, edge_index, W1, b1, W2, b2):
    raise NotImplementedError("write your pallas kernel here")


# speedup vs baseline: 21.3047x; 21.3047x over previous
import jax
import jax.numpy as jnp
from jax import lax
from jax.experimental import pallas as pl
from jax.experimental.pallas import tpu as pltpu
from jax.experimental.pallas import tpu_sc as plsc

N = 10000
E = 320000
TS = 128
DIMS = 4
HID = 256
OUT = 128
KSEL = 2
THO = 0.1

NC = 2
NS = 16
NW = NC * NS
EPW = E // NW
CH = 80
NCHUNK = EPW // CH
NPAD = 10240
RPT = NPAD // NS

_mesh = lambda: plsc.VectorSubcoreMesh(core_axis_name="c", subcore_axis_name="s")


def _sc_hist_body(dst_hbm, ones_hbm, zeros_hbm, hist_out,
                  didx, ones_v, hist_s):
    c = lax.axis_index("c")
    s = lax.axis_index("s")
    ebase = (c * NS + s) * EPW
    pltpu.sync_copy(zeros_hbm, hist_s.at[pl.ds(s * RPT, RPT)])
    pltpu.sync_copy(ones_hbm, ones_v)
    plsc.subcore_barrier()

    def body(i, carry):
        base = ebase + i * CH
        pltpu.sync_copy(dst_hbm.at[pl.ds(base, CH)], didx)
        pltpu.sync_copy(ones_v, hist_s.at[didx], add=True)
        return carry

    lax.fori_loop(0, NCHUNK, body, 0)
    plsc.subcore_barrier()
    pltpu.sync_copy(hist_s.at[pl.ds(s * RPT, RPT)],
                    hist_out.at[c, pl.ds(s * RPT, RPT)])


def _sc_hist(dst, ones, zeros):
    return pl.kernel(
        _sc_hist_body,
        out_type=jax.ShapeDtypeStruct((NC, NPAD), jnp.float32),
        mesh=_mesh(),
        scratch_types=[
            pltpu.VMEM((CH,), jnp.int32),
            pltpu.VMEM((CH,), jnp.float32),
            pltpu.VMEM_SHARED((NPAD,), jnp.float32),
        ],
    )(dst, ones, zeros)


def _sc_agg_body(gs_hbm, dflat_hbm, src_hbm, dst_hbm, zeros128_hbm, zeros_hbm,
                 p_out, t_out,
                 sidx, didx, rows_v, dvals_v, d_v,
                 sem, p_s, t_s):
    c = lax.axis_index("c")
    s = lax.axis_index("s")
    ebase = (c * NS + s) * EPW
    pltpu.sync_copy(zeros128_hbm, p_s.at[pl.ds(s * RPT, RPT)])
    pltpu.sync_copy(zeros_hbm, t_s.at[pl.ds(s * RPT, RPT)])
    pltpu.sync_copy(dflat_hbm, d_v)
    plsc.subcore_barrier()

    def body(i, carry):
        base = ebase + i * CH
        pltpu.sync_copy(src_hbm.at[pl.ds(base, CH)], sidx)
        pltpu.sync_copy(dst_hbm.at[pl.ds(base, CH)], didx)
        pltpu.async_copy(gs_hbm.at[sidx], rows_v, sem).wait()
        pltpu.sync_copy(rows_v, p_s.at[didx], add=True)
        for k in range(CH // 16):
            dvec = didx[pl.ds(k * 16, 16)]
            dvals_v[pl.ds(k * 16, 16)] = plsc.load_gather(d_v, [dvec])
        pltpu.sync_copy(dvals_v, t_s.at[sidx], add=True)
        return carry

    lax.fori_loop(0, NCHUNK, body, 0)
    plsc.subcore_barrier()
    pltpu.sync_copy(p_s.at[pl.ds(s * RPT, RPT)],
                    p_out.at[c, pl.ds(s * RPT, RPT)])
    pltpu.sync_copy(t_s.at[pl.ds(s * RPT, RPT)],
                    t_out.at[c, pl.ds(s * RPT, RPT)])


def _sc_agg(gs, dflat, src, dst, zeros128, zeros):
    return pl.kernel(
        _sc_agg_body,
        out_type=(
            jax.ShapeDtypeStruct((NC, NPAD, TS), jnp.float32),
            jax.ShapeDtypeStruct((NC, NPAD), jnp.float32),
        ),
        mesh=_mesh(),
        compiler_params=pltpu.CompilerParams(needs_layout_passes=False),
        scratch_types=[
            pltpu.VMEM((CH,), jnp.int32),
            pltpu.VMEM((CH,), jnp.int32),
            pltpu.VMEM((CH, TS), jnp.float32),
            pltpu.VMEM((CH,), jnp.float32),
            pltpu.VMEM((NPAD,), jnp.float32),
            pltpu.SemaphoreType.DMA,
            pltpu.VMEM_SHARED((NPAD, TS), jnp.float32),
            pltpu.VMEM_SHARED((NPAD,), jnp.float32),
        ],
    )(gs, dflat, src, dst, zeros128, zeros)


_RB = 2048


def _tc_prep_body(x_ref, h_ref, gs_ref, d1_ref):
    xb = x_ref[...]
    r = lax.broadcasted_iota(jnp.int32, (TS * DIMS, TS), 0)
    col = lax.broadcasted_iota(jnp.int32, (TS * DIMS, TS), 1)
    sel = (r == DIMS * col + KSEL).astype(jnp.float32)
    g = jnp.dot(xb, sel, preferred_element_type=jnp.float32)
    deg = h_ref[0] + h_ref[1] + 1.0
    d = lax.rsqrt(deg)
    gs_ref[...] = g * d
    d1_ref[...] = d


def _tc_prep(x2, hist3):
    nblk = NPAD // _RB
    return pl.pallas_call(
        _tc_prep_body,
        grid=(nblk,),
        in_specs=[
            pl.BlockSpec((_RB, TS * DIMS), lambda i: (i, 0)),
            pl.BlockSpec((NC, _RB, 1), lambda i: (0, i, 0)),
        ],
        out_specs=[
            pl.BlockSpec((_RB, TS), lambda i: (i, 0)),
            pl.BlockSpec((_RB, 1), lambda i: (i, 0)),
        ],
        out_shape=[
            jax.ShapeDtypeStruct((NPAD, TS), jnp.float32),
            jax.ShapeDtypeStruct((NPAD, 1), jnp.float32),
        ],
    )(x2, hist3)


def _tc_final_body(p_ref, gs_ref, d1_ref, t3_ref, w1_ref, b1_ref,
                   w2_ref, b2_ref, out_ref, acc):
    i = pl.program_id(0)
    d = d1_ref[...]
    ag = (p_ref[0] + p_ref[1] + gs_ref[...]) * d
    h = jnp.dot(ag, w1_ref[...], preferred_element_type=jnp.float32)
    relu1 = jnp.maximum(h + b1_ref[...], 0.0)
    row = i * _RB + lax.broadcasted_iota(jnp.int32, (_RB, 1), 0)
    relu1 = jnp.where(row < N, relu1, 0.0)
    t = t3_ref[0] + t3_ref[1]
    cvec = d * (t + d)
    part = lax.dot_general(cvec, relu1, (((0,), (0,)), ((), ())),
                           preferred_element_type=jnp.float32)

    @pl.when(i == 0)
    def _():
        acc[...] = jnp.zeros((1, HID), jnp.float32)

    acc[...] += part

    @pl.when(i == pl.num_programs(0) - 1)
    def _():
        v = acc[...]
        m = jnp.dot(v, w2_ref[...],
                    preferred_element_type=jnp.float32) / float(N) + b2_ref[...]
        o = jax.nn.sigmoid(m)
        ot = jnp.where(o > THO, o - THO,
                       jnp.where(o < -THO, o + THO, 0.0))
        out_ref[...] = jnp.sign(ot) * o


def _tc_final(p2, gs, d1, t3, W1, b1, W2, b2):
    nblk = NPAD // _RB
    return pl.pallas_call(
        _tc_final_body,
        grid=(nblk,),
        in_specs=[
            pl.BlockSpec((NC, _RB, TS), lambda i: (0, i, 0)),
            pl.BlockSpec((_RB, TS), lambda i: (i, 0)),
            pl.BlockSpec((_RB, 1), lambda i: (i, 0)),
            pl.BlockSpec((NC, _RB, 1), lambda i: (0, i, 0)),
            pl.BlockSpec((TS, HID), lambda i: (0, 0)),
            pl.BlockSpec((1, HID), lambda i: (0, 0)),
            pl.BlockSpec((HID, OUT), lambda i: (0, 0)),
            pl.BlockSpec((1, OUT), lambda i: (0, 0)),
        ],
        out_specs=pl.BlockSpec((1, OUT), lambda i: (0, 0)),
        out_shape=jax.ShapeDtypeStruct((1, OUT), jnp.float32),
        scratch_shapes=[pltpu.VMEM((1, HID), jnp.float32)],
    )(p2, gs, d1, t3, W1, b1, W2, b2)


def kernel(x, edge_index, W1, b1, W2, b2):
    src = edge_index[0]
    dst = edge_index[1]
    x2 = x.reshape(N, TS * DIMS)

    ones = jnp.ones((CH,), jnp.float32)
    zeros = jnp.zeros((RPT,), jnp.float32)
    zeros128 = jnp.zeros((RPT, TS), jnp.float32)

    hist = _sc_hist(dst, ones, zeros)
    gs, d1 = _tc_prep(x2, hist.reshape(NC, NPAD, 1))
    p2, t = _sc_agg(gs, d1.reshape(NPAD), src, dst, zeros128, zeros)
    out = _tc_final(p2, gs, d1, t.reshape(NC, NPAD, 1),
                    W1, b1.reshape(1, HID), W2, b2.reshape(1, OUT))
    return out.reshape(OUT, 1)


# speedup vs baseline: 39.8186x; 1.8690x over previous
import jax
import jax.numpy as jnp
from jax import lax
from jax.experimental import pallas as pl
from jax.experimental.pallas import tpu as pltpu
from jax.experimental.pallas import tpu_sc as plsc

N = 10000
E = 320000
TS = 128
DIMS = 4
HID = 256
OUT = 128
KSEL = 2
THO = 0.1

NC = 2
NS = 16
NW = NC * NS
EPW = E // NW
CH = 125
NCHUNK = EPW // CH
GC = 8
NGRP = NCHUNK // GC
NPAD = 10240
RPT = NPAD // NS

_mesh = lambda: plsc.VectorSubcoreMesh(core_axis_name="c", subcore_axis_name="s")


def _sc_hist_body(dst3_hbm, ones_hbm, zeros_hbm, hist_out,
                  didx_all, ones_v, hsem, hist_s):
    c = lax.axis_index("c")
    s = lax.axis_index("s")
    w = c * NS + s
    pltpu.sync_copy(zeros_hbm, hist_s.at[pl.ds(s * RPT, RPT)])
    pltpu.sync_copy(ones_hbm, ones_v)
    pltpu.sync_copy(dst3_hbm.at[w], didx_all)
    plsc.subcore_barrier()

    def fire(i, carry):
        pltpu.async_copy(ones_v, hist_s.at[didx_all.at[i]], hsem, add=True)
        return carry

    lax.fori_loop(0, NCHUNK, fire, 0)

    def drain(i, carry):
        pltpu.make_async_copy(ones_v, hist_s.at[didx_all.at[0]], hsem).wait()
        return carry

    lax.fori_loop(0, NCHUNK, drain, 0)
    plsc.subcore_barrier()
    pltpu.sync_copy(hist_s.at[pl.ds(s * RPT, RPT)],
                    hist_out.at[c, pl.ds(s * RPT, RPT)])


def _sc_hist(dst3, ones, zeros):
    return pl.kernel(
        _sc_hist_body,
        out_type=jax.ShapeDtypeStruct((NC, NPAD), jnp.float32),
        mesh=_mesh(),
        scratch_types=[
            pltpu.VMEM((NCHUNK, CH), jnp.int32),
            pltpu.VMEM((CH,), jnp.float32),
            pltpu.SemaphoreType.DMA,
            pltpu.VMEM_SHARED((NPAD,), jnp.float32),
        ],
    )(dst3, ones, zeros)


def _sc_agg_body(gs_hbm, dflat_hbm, src3_hbm, dst3_hbm, zeros128_hbm, zeros_hbm,
                 p_out, t_out,
                 sg, dg, rows2, dvals2,
                 gsem, dsem, isem, p_s, t_s):
    c = lax.axis_index("c")
    s = lax.axis_index("s")
    w = c * NS + s
    pltpu.sync_copy(zeros128_hbm, p_s.at[pl.ds(s * RPT, RPT)])
    pltpu.sync_copy(zeros_hbm, t_s.at[pl.ds(s * RPT, RPT)])
    pltpu.sync_copy(src3_hbm.at[w, pl.ds(0, GC)], sg.at[0])
    pltpu.sync_copy(dst3_hbm.at[w, pl.ds(0, GC)], dg.at[0])
    plsc.subcore_barrier()
    pltpu.async_copy(src3_hbm.at[w, pl.ds(GC, GC)], sg.at[1], isem)
    pltpu.async_copy(dst3_hbm.at[w, pl.ds(GC, GC)], dg.at[1], isem)
    pltpu.async_copy(gs_hbm.at[sg.at[0, 0]], rows2.at[0], gsem)
    pltpu.async_copy(dflat_hbm.at[dg.at[0, 0]], dvals2.at[0], dsem)

    def gbody(gg, carry):
        for gb in range(2):
            g = gg * 2 + gb
            for j in range(GC):
                i = g * GC + j
                bj = j % 2
                pltpu.make_async_copy(gs_hbm.at[sg.at[gb, j]],
                                      rows2.at[bj], gsem).wait()
                pltpu.make_async_copy(dflat_hbm.at[dg.at[gb, j]],
                                      dvals2.at[bj], dsem).wait()
                if j + 1 < GC:
                    pltpu.async_copy(gs_hbm.at[sg.at[gb, j + 1]],
                                     rows2.at[1 - bj], gsem)
                    pltpu.async_copy(dflat_hbm.at[dg.at[gb, j + 1]],
                                     dvals2.at[1 - bj], dsem)
                else:
                    @pl.when(g + 1 < NGRP)
                    def _():
                        pltpu.make_async_copy(src3_hbm.at[w, pl.ds(0, GC)],
                                              sg.at[1 - gb], isem).wait()
                        pltpu.make_async_copy(dst3_hbm.at[w, pl.ds(0, GC)],
                                              dg.at[1 - gb], isem).wait()
                        pltpu.async_copy(gs_hbm.at[sg.at[1 - gb, 0]],
                                         rows2.at[1 - bj], gsem)
                        pltpu.async_copy(dflat_hbm.at[dg.at[1 - gb, 0]],
                                         dvals2.at[1 - bj], dsem)

                pltpu.sync_copy(rows2.at[bj], p_s.at[dg.at[gb, j]], add=True)
                pltpu.sync_copy(dvals2.at[bj], t_s.at[sg.at[gb, j]], add=True)

            @pl.when(g + 2 < NGRP)
            def _():
                pltpu.async_copy(src3_hbm.at[w, pl.ds((g + 2) * GC, GC)],
                                 sg.at[gb], isem)
                pltpu.async_copy(dst3_hbm.at[w, pl.ds((g + 2) * GC, GC)],
                                 dg.at[gb], isem)
        return carry

    lax.fori_loop(0, NGRP // 2, gbody, 0)
    plsc.subcore_barrier()
    pltpu.sync_copy(p_s.at[pl.ds(s * RPT, RPT)],
                    p_out.at[c, pl.ds(s * RPT, RPT)])
    pltpu.sync_copy(t_s.at[pl.ds(s * RPT, RPT)],
                    t_out.at[c, pl.ds(s * RPT, RPT)])


def _sc_agg(gs, dflat, src3, dst3, zeros128, zeros):
    return pl.kernel(
        _sc_agg_body,
        out_type=(
            jax.ShapeDtypeStruct((NC, NPAD, TS), jnp.float32),
            jax.ShapeDtypeStruct((NC, NPAD), jnp.float32),
        ),
        mesh=_mesh(),
        compiler_params=pltpu.CompilerParams(needs_layout_passes=False),
        scratch_types=[
            pltpu.VMEM((2, GC, CH), jnp.int32),
            pltpu.VMEM((2, GC, CH), jnp.int32),
            pltpu.VMEM((2, CH, TS), jnp.float32),
            pltpu.VMEM((2, CH), jnp.float32),
            pltpu.SemaphoreType.DMA,
            pltpu.SemaphoreType.DMA,
            pltpu.SemaphoreType.DMA,
            pltpu.VMEM_SHARED((NPAD, TS), jnp.float32),
            pltpu.VMEM_SHARED((NPAD,), jnp.float32),
        ],
    )(gs, dflat, src3, dst3, zeros128, zeros)


_RB = 2048


def _tc_prep_body(x_ref, h_ref, gs_ref, d1_ref):
    xb = x_ref[...]
    r = lax.broadcasted_iota(jnp.int32, (TS * DIMS, TS), 0)
    col = lax.broadcasted_iota(jnp.int32, (TS * DIMS, TS), 1)
    sel = (r == DIMS * col + KSEL).astype(jnp.float32)
    g = jnp.dot(xb, sel, preferred_element_type=jnp.float32)
    deg = h_ref[0] + h_ref[1] + 1.0
    d = lax.rsqrt(deg)
    gs_ref[...] = g * d
    d1_ref[...] = d


def _tc_prep(x2, hist3):
    nblk = NPAD // _RB
    return pl.pallas_call(
        _tc_prep_body,
        grid=(nblk,),
        in_specs=[
            pl.BlockSpec((_RB, TS * DIMS), lambda i: (i, 0)),
            pl.BlockSpec((NC, _RB, 1), lambda i: (0, i, 0)),
        ],
        out_specs=[
            pl.BlockSpec((_RB, TS), lambda i: (i, 0)),
            pl.BlockSpec((_RB, 1), lambda i: (i, 0)),
        ],
        out_shape=[
            jax.ShapeDtypeStruct((NPAD, TS), jnp.float32),
            jax.ShapeDtypeStruct((NPAD, 1), jnp.float32),
        ],
    )(x2, hist3)


def _tc_final_body(p_ref, gs_ref, d1_ref, t3_ref, w1_ref, b1_ref,
                   w2_ref, b2_ref, out_ref, acc):
    i = pl.program_id(0)
    d = d1_ref[...]
    ag = (p_ref[0] + p_ref[1] + gs_ref[...]) * d
    h = jnp.dot(ag, w1_ref[...], preferred_element_type=jnp.float32)
    relu1 = jnp.maximum(h + b1_ref[...], 0.0)
    row = i * _RB + lax.broadcasted_iota(jnp.int32, (_RB, 1), 0)
    relu1 = jnp.where(row < N, relu1, 0.0)
    t = t3_ref[0] + t3_ref[1]
    cvec = d * (t + d)
    part = lax.dot_general(cvec, relu1, (((0,), (0,)), ((), ())),
                           preferred_element_type=jnp.float32)

    @pl.when(i == 0)
    def _():
        acc[...] = jnp.zeros((1, HID), jnp.float32)

    acc[...] += part

    @pl.when(i == pl.num_programs(0) - 1)
    def _():
        v = acc[...]
        m = jnp.dot(v, w2_ref[...],
                    preferred_element_type=jnp.float32) / float(N) + b2_ref[...]
        o = jax.nn.sigmoid(m)
        ot = jnp.where(o > THO, o - THO,
                       jnp.where(o < -THO, o + THO, 0.0))
        out_ref[...] = jnp.sign(ot) * o


def _tc_final(p2, gs, d1, t3, W1, b1, W2, b2):
    nblk = NPAD // _RB
    return pl.pallas_call(
        _tc_final_body,
        grid=(nblk,),
        in_specs=[
            pl.BlockSpec((NC, _RB, TS), lambda i: (0, i, 0)),
            pl.BlockSpec((_RB, TS), lambda i: (i, 0)),
            pl.BlockSpec((_RB, 1), lambda i: (i, 0)),
            pl.BlockSpec((NC, _RB, 1), lambda i: (0, i, 0)),
            pl.BlockSpec((TS, HID), lambda i: (0, 0)),
            pl.BlockSpec((1, HID), lambda i: (0, 0)),
            pl.BlockSpec((HID, OUT), lambda i: (0, 0)),
            pl.BlockSpec((1, OUT), lambda i: (0, 0)),
        ],
        out_specs=pl.BlockSpec((1, OUT), lambda i: (0, 0)),
        out_shape=jax.ShapeDtypeStruct((1, OUT), jnp.float32),
        scratch_shapes=[pltpu.VMEM((1, HID), jnp.float32)],
    )(p2, gs, d1, t3, W1, b1, W2, b2)


def kernel(x, edge_index, W1, b1, W2, b2):
    src3 = edge_index[0].reshape(NW, NCHUNK, CH)
    dst3 = edge_index[1].reshape(NW, NCHUNK, CH)
    x2 = x.reshape(N, TS * DIMS)

    ones = jnp.ones((CH,), jnp.float32)
    zeros = jnp.zeros((RPT,), jnp.float32)
    zeros128 = jnp.zeros((RPT, TS), jnp.float32)

    hist = _sc_hist(dst3, ones, zeros)
    gs, d1 = _tc_prep(x2, hist.reshape(NC, NPAD, 1))
    p2, t = _sc_agg(gs, d1.reshape(NPAD), src3, dst3, zeros128, zeros)
    out = _tc_final(p2, gs, d1, t.reshape(NC, NPAD, 1),
                    W1, b1.reshape(1, HID), W2, b2.reshape(1, OUT))
    return out.reshape(OUT, 1)


# speedup vs baseline: 46.0620x; 1.1568x over previous
import jax
import jax.numpy as jnp
from jax import lax
from jax.experimental import pallas as pl
from jax.experimental.pallas import tpu as pltpu
from jax.experimental.pallas import tpu_sc as plsc

N = 10000
E = 320000
TS = 128
DIMS = 4
HID = 256
OUT = 128
KSEL = 2
THO = 0.1

NC = 2
NS = 16
NW = NC * NS
EPW = E // NW
CH = 125
NCHUNK = EPW // CH
GC = 8
NGRP = NCHUNK // GC
NPAD = 10240
RPT = NPAD // NS

_mesh = lambda: plsc.VectorSubcoreMesh(core_axis_name="c", subcore_axis_name="s",
                                       num_cores=NC, num_subcores=NS)


def _sc_hist_body(dst3_hbm, ones_hbm, zeros_hbm, hist_out,
                  didx_all, ones_v, hsem, hist_s):
    c = lax.axis_index("c")
    s = lax.axis_index("s")
    w = c * NS + s
    pltpu.sync_copy(zeros_hbm, hist_s.at[pl.ds(s * RPT, RPT)])
    pltpu.sync_copy(ones_hbm, ones_v)
    pltpu.sync_copy(dst3_hbm.at[w], didx_all)
    plsc.subcore_barrier()

    def fire(i, carry):
        pltpu.async_copy(ones_v, hist_s.at[didx_all.at[i]], hsem, add=True)
        return carry

    lax.fori_loop(0, NCHUNK, fire, 0)

    def drain(i, carry):
        pltpu.make_async_copy(ones_v, hist_s.at[didx_all.at[0]], hsem).wait()
        return carry

    lax.fori_loop(0, NCHUNK, drain, 0)
    plsc.subcore_barrier()
    pltpu.sync_copy(hist_s.at[pl.ds(s * RPT, RPT)],
                    hist_out.at[c, pl.ds(s * RPT, RPT)])


def _sc_hist(dst3, ones, zeros):
    return pl.kernel(
        _sc_hist_body,
        out_type=jax.ShapeDtypeStruct((NC, NPAD), jnp.float32),
        mesh=_mesh(),
        scratch_types=[
            pltpu.VMEM((NCHUNK, CH), jnp.int32),
            pltpu.VMEM((CH,), jnp.float32),
            pltpu.SemaphoreType.DMA,
            pltpu.VMEM_SHARED((NPAD,), jnp.float32),
        ],
    )(dst3, ones, zeros)


def _sc_agg_body(gs_hbm, dflat_hbm, src3_hbm, dst3_hbm, zeros128_hbm, zeros_hbm,
                 p_out, t_out,
                 sg, dg, rows2, dvals2,
                 gsem, dsem, isem, p_s, t_s):
    c = lax.axis_index("c")
    s = lax.axis_index("s")
    w = c * NS + s
    pltpu.sync_copy(zeros128_hbm, p_s.at[pl.ds(s * RPT, RPT)])
    pltpu.sync_copy(zeros_hbm, t_s.at[pl.ds(s * RPT, RPT)])
    pltpu.sync_copy(src3_hbm.at[w, pl.ds(0, GC)], sg.at[0])
    pltpu.sync_copy(dst3_hbm.at[w, pl.ds(0, GC)], dg.at[0])
    plsc.subcore_barrier()
    pltpu.async_copy(src3_hbm.at[w, pl.ds(GC, GC)], sg.at[1], isem)
    pltpu.async_copy(dst3_hbm.at[w, pl.ds(GC, GC)], dg.at[1], isem)
    pltpu.async_copy(gs_hbm.at[sg.at[0, 0]], rows2.at[0], gsem)
    pltpu.async_copy(dflat_hbm.at[dg.at[0, 0]], dvals2.at[0], dsem)

    def gbody(gg, carry):
        for gb in range(2):
            g = gg * 2 + gb
            for j in range(GC):
                i = g * GC + j
                bj = j % 2
                pltpu.make_async_copy(gs_hbm.at[sg.at[gb, j]],
                                      rows2.at[bj], gsem).wait()
                pltpu.make_async_copy(dflat_hbm.at[dg.at[gb, j]],
                                      dvals2.at[bj], dsem).wait()
                if j + 1 < GC:
                    pltpu.async_copy(gs_hbm.at[sg.at[gb, j + 1]],
                                     rows2.at[1 - bj], gsem)
                    pltpu.async_copy(dflat_hbm.at[dg.at[gb, j + 1]],
                                     dvals2.at[1 - bj], dsem)
                else:
                    @pl.when(g + 1 < NGRP)
                    def _():
                        pltpu.make_async_copy(src3_hbm.at[w, pl.ds(0, GC)],
                                              sg.at[1 - gb], isem).wait()
                        pltpu.make_async_copy(dst3_hbm.at[w, pl.ds(0, GC)],
                                              dg.at[1 - gb], isem).wait()
                        pltpu.async_copy(gs_hbm.at[sg.at[1 - gb, 0]],
                                         rows2.at[1 - bj], gsem)
                        pltpu.async_copy(dflat_hbm.at[dg.at[1 - gb, 0]],
                                         dvals2.at[1 - bj], dsem)

                pltpu.sync_copy(rows2.at[bj], p_s.at[dg.at[gb, j]], add=True)
                pltpu.sync_copy(dvals2.at[bj], t_s.at[sg.at[gb, j]], add=True)

            @pl.when(g + 2 < NGRP)
            def _():
                pltpu.async_copy(src3_hbm.at[w, pl.ds((g + 2) * GC, GC)],
                                 sg.at[gb], isem)
                pltpu.async_copy(dst3_hbm.at[w, pl.ds((g + 2) * GC, GC)],
                                 dg.at[gb], isem)
        return carry

    lax.fori_loop(0, NGRP // 2, gbody, 0)
    plsc.subcore_barrier()
    pltpu.sync_copy(p_s.at[pl.ds(s * RPT, RPT)],
                    p_out.at[c, pl.ds(s * RPT, RPT)])
    pltpu.sync_copy(t_s.at[pl.ds(s * RPT, RPT)],
                    t_out.at[c, pl.ds(s * RPT, RPT)])


def _sc_agg(gs, dflat, src3, dst3, zeros128, zeros):
    return pl.kernel(
        _sc_agg_body,
        out_type=(
            jax.ShapeDtypeStruct((NC, NPAD, TS), jnp.float32),
            jax.ShapeDtypeStruct((NC, NPAD), jnp.float32),
        ),
        mesh=_mesh(),
        compiler_params=pltpu.CompilerParams(needs_layout_passes=False),
        scratch_types=[
            pltpu.VMEM((2, GC, CH), jnp.int32),
            pltpu.VMEM((2, GC, CH), jnp.int32),
            pltpu.VMEM((2, CH, TS), jnp.float32),
            pltpu.VMEM((2, CH), jnp.float32),
            pltpu.SemaphoreType.DMA,
            pltpu.SemaphoreType.DMA,
            pltpu.SemaphoreType.DMA,
            pltpu.VMEM_SHARED((NPAD, TS), jnp.float32),
            pltpu.VMEM_SHARED((NPAD,), jnp.float32),
        ],
    )(gs, dflat, src3, dst3, zeros128, zeros)


_RB = 2048


def _tc_prep_body(g_ref, h_ref, gs_ref, d1_ref):
    deg = h_ref[0] + h_ref[1] + 1.0
    d = lax.rsqrt(deg)
    gs_ref[...] = g_ref[...] * d
    d1_ref[...] = d


def _tc_prep(g, hist3):
    nblk = NPAD // _RB
    return pl.pallas_call(
        _tc_prep_body,
        grid=(nblk,),
        in_specs=[
            pl.BlockSpec((_RB, TS), lambda i: (i, 0)),
            pl.BlockSpec((NC, _RB, 1), lambda i: (0, i, 0)),
        ],
        out_specs=[
            pl.BlockSpec((_RB, TS), lambda i: (i, 0)),
            pl.BlockSpec((_RB, 1), lambda i: (i, 0)),
        ],
        out_shape=[
            jax.ShapeDtypeStruct((NPAD, TS), jnp.float32),
            jax.ShapeDtypeStruct((NPAD, 1), jnp.float32),
        ],
    )(g, hist3)


def _tc_final_body(p_ref, gs_ref, d1_ref, t3_ref, w1_ref, b1_ref,
                   w2_ref, b2_ref, out_ref, acc):
    i = pl.program_id(0)
    d = d1_ref[...]
    ag = (p_ref[0] + p_ref[1] + gs_ref[...]) * d
    h = jnp.dot(ag, w1_ref[...], preferred_element_type=jnp.float32)
    relu1 = jnp.maximum(h + b1_ref[...], 0.0)
    row = i * _RB + lax.broadcasted_iota(jnp.int32, (_RB, 1), 0)
    relu1 = jnp.where(row < N, relu1, 0.0)
    t = t3_ref[0] + t3_ref[1]
    cvec = d * (t + d)
    part = lax.dot_general(cvec, relu1, (((0,), (0,)), ((), ())),
                           preferred_element_type=jnp.float32)

    @pl.when(i == 0)
    def _():
        acc[...] = jnp.zeros((1, HID), jnp.float32)

    acc[...] += part

    @pl.when(i == pl.num_programs(0) - 1)
    def _():
        v = acc[...]
        m = jnp.dot(v, w2_ref[...],
                    preferred_element_type=jnp.float32) / float(N) + b2_ref[...]
        o = jax.nn.sigmoid(m)
        ot = jnp.where(o > THO, o - THO,
                       jnp.where(o < -THO, o + THO, 0.0))
        out_ref[...] = jnp.sign(ot) * o


def _tc_final(p2, gs, d1, t3, W1, b1, W2, b2):
    nblk = NPAD // _RB
    return pl.pallas_call(
        _tc_final_body,
        grid=(nblk,),
        in_specs=[
            pl.BlockSpec((NC, _RB, TS), lambda i: (0, i, 0)),
            pl.BlockSpec((_RB, TS), lambda i: (i, 0)),
            pl.BlockSpec((_RB, 1), lambda i: (i, 0)),
            pl.BlockSpec((NC, _RB, 1), lambda i: (0, i, 0)),
            pl.BlockSpec((TS, HID), lambda i: (0, 0)),
            pl.BlockSpec((1, HID), lambda i: (0, 0)),
            pl.BlockSpec((HID, OUT), lambda i: (0, 0)),
            pl.BlockSpec((1, OUT), lambda i: (0, 0)),
        ],
        out_specs=pl.BlockSpec((1, OUT), lambda i: (0, 0)),
        out_shape=jax.ShapeDtypeStruct((1, OUT), jnp.float32),
        scratch_shapes=[pltpu.VMEM((1, HID), jnp.float32)],
    )(p2, gs, d1, t3, W1, b1, W2, b2)


def kernel(x, edge_index, W1, b1, W2, b2):
    src3 = edge_index[0].reshape(NW, NCHUNK, CH)
    dst3 = edge_index[1].reshape(NW, NCHUNK, CH)
    g = x[:, :, KSEL]

    ones = jnp.ones((CH,), jnp.float32)
    zeros = jnp.zeros((RPT,), jnp.float32)
    zeros128 = jnp.zeros((RPT, TS), jnp.float32)

    hist = _sc_hist(dst3, ones, zeros)
    gs, d1 = _tc_prep(g, hist.reshape(NC, NPAD, 1))
    p2, t = _sc_agg(gs, d1.reshape(NPAD), src3, dst3, zeros128, zeros)
    out = _tc_final(p2, gs, d1, t.reshape(NC, NPAD, 1),
                    W1, b1.reshape(1, HID), W2, b2.reshape(1, OUT))
    return out.reshape(OUT, 1)
